# baseline (speedup 1.0000x reference)
import dataclasses
import functools

import jax
import jax.numpy as jnp
from jax import lax
from jax.experimental import pallas as pl
from jax.experimental.pallas import tpu as pltpu
from jax.experimental.pallas import tpu_sc as plsc

D = 64
DW = D // 2
L = 16
NC = 2
NS = 16
NW = NC * NS
W = 32
GCHUNK = 128
KSPLIT = 10
PF = plsc.PackFormat.INTERLEAVED


def _compiler_params(tc_tiling):
    cp = pltpu.CompilerParams()
    f = pltpu.CompilerParams.__dataclass_fields__
    if "needs_layout_passes" in f:
        cp = dataclasses.replace(cp, needs_layout_passes=False)
    if "use_tc_tiling_on_sc" in f:
        cp = dataclasses.replace(cp, use_tc_tiling_on_sc=tc_tiling)
    return cp


def _sc_linearize(in_t, out_t, tail_in, tail_out, V):
    NB = 2
    CW = 256
    NQ_FULL = V // CW
    NJ = (NQ_FULL // NW) & ~(NB - 1)
    NQ_MAIN = NJ * NW
    N_EXTRA = NQ_FULL - NQ_MAIN
    TAIL = V - NQ_FULL * CW

    mesh = plsc.VectorSubcoreMesh(core_axis_name="c", subcore_axis_name="s")

    @functools.partial(
        pl.kernel,
        compiler_params=_compiler_params(True),
        out_type=(
            jax.ShapeDtypeStruct((V * DW,), jnp.int32),
            jax.ShapeDtypeStruct((V * DW,), jnp.int32),
        ),
        mesh=mesh,
        scratch_types=(
            [pltpu.VMEM((D, CW), jnp.float32)] * 2
            + [pltpu.VMEM((CW * DW,), jnp.int32)] * 2
            + [pltpu.VMEM((TAIL * D,), jnp.float32)]
            + [pltpu.VMEM((TAIL * DW,), jnp.int32)]
            + [pltpu.SemaphoreType.DMA] * 4
        ),
    )
    def a(in_hbm, out_hbm, tin_hbm, tout_hbm, lin_in, lin_out, *scratch):
        wid = lax.axis_index("s") * NC + lax.axis_index("c")
        bins, bouts = scratch[0:2], scratch[2:4]
        btail, btailw = scratch[4], scratch[5]
        sis, sos = scratch[6:8], scratch[8:10]
        iota = lax.iota(jnp.int32, L)
        iota2 = iota * 2
        cgrp = [(j * 2 * L + iota2, j * 2 * L + iota2 + 1, j * L + iota)
                for j in range(D // (2 * L))]

        def transpose_block(src, dst, ncols):
            @plsc.parallel_loop(0, ncols, L, unroll=4)
            def _(m0):
                for s in range(L):
                    perm = (iota + s) & (L - 1)
                    mcol = m0 + perm
                    mcolw = mcol * DW
                    for ca, cb, cw in cgrp:
                        va = plsc.load_gather(src, [ca, mcol])
                        vb = plsc.load_gather(src, [cb, mcol])
                        w = plsc.bitcast(plsc.pack(va, vb, format=PF),
                                         jnp.int32)
                        plsc.store_scatter(dst, [mcolw + cw], w)

        def issue_in(src, s, q):
            pltpu.async_copy(
                src.at[:, pl.ds(pl.multiple_of(q * CW, 128), CW)],
                bins[s], sis[s])

        def wait_in(src, s):
            pltpu.make_async_copy(src.at[:, pl.ds(0, CW)], bins[s],
                                  sis[s]).wait()

        def issue_out(dst, s, q):
            pltpu.async_copy(
                bouts[s],
                dst.at[pl.ds(pl.multiple_of(q * CW * DW, 128 * DW),
                             CW * DW)], sos[s])

        def wait_out(dst, s):
            pltpu.make_async_copy(bouts[s], dst.at[pl.ds(0, CW * DW)],
                                  sos[s]).wait()

        for src, dst, tsrc in ((in_hbm, lin_in, tin_hbm),
                               (out_hbm, lin_out, tout_hbm)):
            for s in range(NB):
                issue_in(src, s, wid + s * NW)

            @pl.loop(0, NJ, step=NB)
            def _(j, src=src, dst=dst):
                for s in range(NB):
                    jj = j + s
                    q = jj * NW + wid
                    wait_in(src, s)

                    @pl.when(jj >= NB)
                    def _():
                        wait_out(dst, s)

                    transpose_block(bins[s], bouts[s], CW)
                    issue_out(dst, s, q)

                    @pl.when(jj + NB < NJ)
                    def _():
                        issue_in(src, s, q + NB * NW)

            for s in range(NB):
                wait_out(dst, s)

            @pl.when(wid < N_EXTRA)
            def _(src=src, dst=dst):
                q = NQ_MAIN + wid
                issue_in(src, 0, q)
                wait_in(src, 0)
                transpose_block(bins[0], bouts[0], CW)
                issue_out(dst, 0, q)
                wait_out(dst, 0)

            @pl.when(wid == N_EXTRA)
            def _(dst=dst, tsrc=tsrc):
                pltpu.sync_copy(tsrc, btail)

                @pl.loop(0, TAIL * D // (2 * L))
                def _(i):
                    va = plsc.load_gather(btail, [i * 2 * L + iota2])
                    vb = plsc.load_gather(btail, [i * 2 * L + iota2 + 1])
                    w = plsc.bitcast(plsc.pack(va, vb, format=PF),
                                     jnp.int32)
                    btailw[pl.ds(i * L, L)] = w

                pltpu.sync_copy(btailw, dst.at[pl.ds(NQ_FULL * CW * DW,
                                                     TAIL * DW)])

    return a(in_t, out_t, tail_in, tail_out)


def _sc_scores(in_lin, out_lin, center, positive, negatives_flat, B, K):
    BPW = B // NW
    NIDX = BPW * K
    NBLK = BPW // W
    NEG_CH = (W * K) // GCHUNK

    mesh = plsc.VectorSubcoreMesh(core_axis_name="c", subcore_axis_name="s")

    @functools.partial(
        pl.kernel,
        compiler_params=_compiler_params(False),
        out_type=(
            jax.ShapeDtypeStruct((B,), jnp.float32),
            jax.ShapeDtypeStruct((B * K,), jnp.float32),
        ),
        mesh=mesh,
        scratch_types=[
            pltpu.VMEM((BPW,), jnp.int32),
            pltpu.VMEM((BPW,), jnp.int32),
            pltpu.VMEM((NIDX,), jnp.int32),
            pltpu.VMEM((W, DW), jnp.int32),
            pltpu.VMEM((W, DW), jnp.int32),
            pltpu.VMEM((W * K, DW), jnp.int32),
            pltpu.VMEM((BPW,), jnp.float32),
            pltpu.VMEM((NIDX,), jnp.float32),
            pltpu.SemaphoreType.DMA,
        ],
    )
    def k(in_hbm, out_hbm, c_hbm, p_hbm, n_hbm, pos_out, neg_out,
          cidx, pidx, nidx, crows, prows, nrows, poss, negs, sem):
        wid = lax.axis_index("s") * NC + lax.axis_index("c")
        base = wid * BPW
        pltpu.sync_copy(c_hbm.at[pl.ds(base, BPW)], cidx)
        pltpu.sync_copy(p_hbm.at[pl.ds(base, BPW)], pidx)
        pltpu.sync_copy(n_hbm.at[pl.ds(base * K, NIDX)], nidx)

        @pl.loop(0, NBLK)
        def _(blk):
            off = blk * W
            cps = [
                pltpu.async_copy(in_hbm.at[cidx.at[pl.ds(off, W)]], crows,
                                 sem),
                pltpu.async_copy(out_hbm.at[pidx.at[pl.ds(off, W)]], prows,
                                 sem),
            ]
            for j in range(NEG_CH):
                cps.append(pltpu.async_copy(
                    out_hbm.at[nidx.at[pl.ds(off * K + j * GCHUNK, GCHUNK)]],
                    nrows.at[pl.ds(j * GCHUNK, GCHUNK)], sem))
            for c in cps:
                c.wait()

            iota = lax.iota(jnp.int32, L)
            for g in range(W // L):
                rows = g * L + iota
                rows_k = rows * K

                for k0 in range(0, K, KSPLIT):
                    kspan = list(range(k0, min(k0 + KSPLIT, K)))
                    nrowv = [rows_k + kk for kk in kspan]

                    def dbody(d, accs, kspan=kspan, nrowv=nrowv):
                        dv = (d + iota) & (DW - 1)
                        cbf = plsc.bitcast(
                            plsc.load_gather(crows, [rows, dv]),
                            jnp.bfloat16)
                        new = []
                        if kspan[0] == 0:
                            pbf = plsc.bitcast(
                                plsc.load_gather(prows, [rows, dv]),
                                jnp.bfloat16)
                            pa, pb = plsc.unpack(
                                cbf * pbf, format=PF,
                                preferred_element_type=jnp.float32)
                            new.append(accs[0] + (pa + pb))
                        for j, kk in enumerate(kspan):
                            nbf = plsc.bitcast(
                                plsc.load_gather(nrows, [nrowv[j], dv]),
                                jnp.bfloat16)
                            pa, pb = plsc.unpack(
                                cbf * nbf, format=PF,
                                preferred_element_type=jnp.float32)
                            new.append(accs[len(new)] + (pa + pb))
                        return tuple(new)

                    n_acc = len(kspan) + (1 if k0 == 0 else 0)
                    init = tuple(jnp.zeros((L,), jnp.float32)
                                 for _ in range(n_acc))
                    res = lax.fori_loop(0, DW, dbody, init)
                    if k0 == 0:
                        poss[pl.ds(off + g * L, L)] = res[0]
                        res = res[1:]
                    for j, kk in enumerate(kspan):
                        negs[pl.ds(kk * BPW + off + g * L, L)] = res[j]

        pltpu.sync_copy(poss, pos_out.at[pl.ds(base, BPW)])
        pltpu.sync_copy(negs, neg_out.at[pl.ds(base * K, NIDX)])

    return k(in_lin, out_lin, center, positive, negatives_flat)


def _tc_loss(pos_s, neg_s, B, K):
    pos2 = pos_s.reshape(B // 128, 128)
    neg2 = neg_s.reshape((B * K) // 128, 128)

    def body(pos_ref, neg_ref, o_ref):
        ps = pos_ref[...]
        ns = neg_ref[...]
        pos_loss = jnp.sum(jnp.maximum(-ps, 0.0)
                           + jnp.log1p(jnp.exp(-jnp.abs(ps))))
        neg_loss = jnp.sum(jnp.maximum(ns, 0.0)
                           + jnp.log1p(jnp.exp(-jnp.abs(ns))))
        o_ref[0, 0] = pos_loss / B + neg_loss / (B * K)

    return pl.pallas_call(
        body,
        out_shape=jax.ShapeDtypeStruct((1, 1), jnp.float32),
        in_specs=[
            pl.BlockSpec(memory_space=pltpu.VMEM),
            pl.BlockSpec(memory_space=pltpu.VMEM),
        ],
        out_specs=pl.BlockSpec(memory_space=pltpu.SMEM),
    )(pos2, neg2)


def kernel(in_emb, out_emb, center, positive, negatives):
    B, K = negatives.shape
    V = in_emb.shape[0]
    ntail = V % 128
    tail_in = in_emb[V - ntail:, :].reshape(ntail * D)
    tail_out = out_emb[V - ntail:, :].reshape(ntail * D)
    lin_in, lin_out = _sc_linearize(in_emb.T, out_emb.T, tail_in, tail_out,
                                    V)
    in2 = lin_in.reshape(V, DW)
    out2 = lin_out.reshape(V, DW)
    center = center.astype(jnp.int32)
    positive = positive.astype(jnp.int32)
    negatives_flat = negatives.astype(jnp.int32).reshape(B * K)
    pos_s, neg_s = _sc_scores(in2, out2, center, positive,
                              negatives_flat, B, K)
    return _tc_loss(pos_s, neg_s, B, K)[0, 0]


# speedup vs baseline: 1.0311x; 1.0311x over previous
import dataclasses
import functools

import jax
import jax.numpy as jnp
from jax import lax
from jax.experimental import pallas as pl
from jax.experimental.pallas import tpu as pltpu
from jax.experimental.pallas import tpu_sc as plsc

D = 64
DW = D // 2
L = 16
NC = 2
NS = 16
NW = NC * NS
W = 32
GCHUNK = 128
KSPLIT = 10
PF = plsc.PackFormat.INTERLEAVED


def _compiler_params(tc_tiling):
    cp = pltpu.CompilerParams()
    f = pltpu.CompilerParams.__dataclass_fields__
    if "needs_layout_passes" in f:
        cp = dataclasses.replace(cp, needs_layout_passes=False)
    if "use_tc_tiling_on_sc" in f:
        cp = dataclasses.replace(cp, use_tc_tiling_on_sc=tc_tiling)
    return cp


def _sc_linearize(in_t, out_t, tail_in, tail_out, V):
    NB = 2
    CW = 256
    NQ_FULL = V // CW
    NJ = (NQ_FULL // NW) & ~(NB - 1)
    NQ_MAIN = NJ * NW
    N_EXTRA = NQ_FULL - NQ_MAIN
    TAIL = V - NQ_FULL * CW

    mesh = plsc.VectorSubcoreMesh(core_axis_name="c", subcore_axis_name="s")

    @functools.partial(
        pl.kernel,
        compiler_params=_compiler_params(True),
        out_type=(
            jax.ShapeDtypeStruct((V * DW,), jnp.int32),
            jax.ShapeDtypeStruct((V * DW,), jnp.int32),
        ),
        mesh=mesh,
        scratch_types=(
            [pltpu.VMEM((D, CW), jnp.float32)] * 2
            + [pltpu.VMEM((CW * DW,), jnp.int32)] * 2
            + [pltpu.VMEM((TAIL * D,), jnp.float32)]
            + [pltpu.VMEM((TAIL * DW,), jnp.int32)]
            + [pltpu.SemaphoreType.DMA] * 4
        ),
    )
    def a(in_hbm, out_hbm, tin_hbm, tout_hbm, lin_in, lin_out, *scratch):
        wid = lax.axis_index("s") * NC + lax.axis_index("c")
        bins, bouts = scratch[0:2], scratch[2:4]
        btail, btailw = scratch[4], scratch[5]
        sis, sos = scratch[6:8], scratch[8:10]
        iota = lax.iota(jnp.int32, L)
        iota2 = iota * 2
        cgrp = [(j * 2 * L + iota2, j * 2 * L + iota2 + 1, j * L + iota)
                for j in range(D // (2 * L))]

        def transpose_block(src, dst, ncols):
            @plsc.parallel_loop(0, ncols, L, unroll=2)
            def _(m0):
                for s in range(L):
                    perm = (iota + s) & (L - 1)
                    mcol = m0 + perm
                    mcolw = mcol * DW
                    for ca, cb, cw in cgrp:
                        va = plsc.load_gather(src, [ca, mcol])
                        vb = plsc.load_gather(src, [cb, mcol])
                        w = plsc.bitcast(plsc.pack(va, vb, format=PF),
                                         jnp.int32)
                        plsc.store_scatter(dst, [mcolw + cw], w)

        def issue_in(src, s, q):
            c0 = pl.multiple_of(q * CW, 128)
            for h in range(2):
                pltpu.async_copy(
                    src.at[pl.ds(h * D // 2, D // 2), pl.ds(c0, CW)],
                    bins[s].at[pl.ds(h * D // 2, D // 2), :], sis[s])

        def wait_in(src, s):
            for h in range(2):
                pltpu.make_async_copy(
                    src.at[pl.ds(0, D // 2), pl.ds(0, CW)],
                    bins[s].at[pl.ds(0, D // 2), :], sis[s]).wait()

        def issue_out(dst, s, q):
            pltpu.async_copy(
                bouts[s],
                dst.at[pl.ds(pl.multiple_of(q * CW * DW, 128 * DW),
                             CW * DW)], sos[s])

        def wait_out(dst, s):
            pltpu.make_async_copy(bouts[s], dst.at[pl.ds(0, CW * DW)],
                                  sos[s]).wait()

        for src, dst, tsrc in ((in_hbm, lin_in, tin_hbm),
                               (out_hbm, lin_out, tout_hbm)):
            for s in range(NB):
                issue_in(src, s, wid + s * NW)

            @pl.loop(0, NJ, step=NB)
            def _(j, src=src, dst=dst):
                for s in range(NB):
                    jj = j + s
                    q = jj * NW + wid
                    wait_in(src, s)

                    @pl.when(jj >= NB)
                    def _():
                        wait_out(dst, s)

                    transpose_block(bins[s], bouts[s], CW)
                    issue_out(dst, s, q)

                    @pl.when(jj + NB < NJ)
                    def _():
                        issue_in(src, s, q + NB * NW)

            for s in range(NB):
                wait_out(dst, s)

            @pl.when(wid < N_EXTRA)
            def _(src=src, dst=dst):
                q = NQ_MAIN + wid
                issue_in(src, 0, q)
                wait_in(src, 0)
                transpose_block(bins[0], bouts[0], CW)
                issue_out(dst, 0, q)
                wait_out(dst, 0)

            @pl.when(wid == N_EXTRA)
            def _(dst=dst, tsrc=tsrc):
                pltpu.sync_copy(tsrc, btail)

                @pl.loop(0, TAIL * D // (2 * L))
                def _(i):
                    va = plsc.load_gather(btail, [i * 2 * L + iota2])
                    vb = plsc.load_gather(btail, [i * 2 * L + iota2 + 1])
                    w = plsc.bitcast(plsc.pack(va, vb, format=PF),
                                     jnp.int32)
                    btailw[pl.ds(i * L, L)] = w

                pltpu.sync_copy(btailw, dst.at[pl.ds(NQ_FULL * CW * DW,
                                                     TAIL * DW)])

    return a(in_t, out_t, tail_in, tail_out)


def _sc_scores(in_lin, out_lin, center, positive, negatives_flat, B, K):
    BPW = B // NW
    NIDX = BPW * K
    NBLK = BPW // W
    NEG_CH = (W * K) // GCHUNK

    mesh = plsc.VectorSubcoreMesh(core_axis_name="c", subcore_axis_name="s")

    @functools.partial(
        pl.kernel,
        compiler_params=_compiler_params(False),
        out_type=(
            jax.ShapeDtypeStruct((B,), jnp.float32),
            jax.ShapeDtypeStruct((B * K,), jnp.float32),
        ),
        mesh=mesh,
        scratch_types=[
            pltpu.VMEM((BPW,), jnp.int32),
            pltpu.VMEM((BPW,), jnp.int32),
            pltpu.VMEM((NIDX,), jnp.int32),
            pltpu.VMEM((W, DW), jnp.int32),
            pltpu.VMEM((W, DW), jnp.int32),
            pltpu.VMEM((W * K, DW), jnp.int32),
            pltpu.VMEM((BPW,), jnp.float32),
            pltpu.VMEM((NIDX,), jnp.float32),
            pltpu.SemaphoreType.DMA,
        ],
    )
    def k(in_hbm, out_hbm, c_hbm, p_hbm, n_hbm, pos_out, neg_out,
          cidx, pidx, nidx, crows, prows, nrows, poss, negs, sem):
        wid = lax.axis_index("s") * NC + lax.axis_index("c")
        base = wid * BPW
        pltpu.sync_copy(c_hbm.at[pl.ds(base, BPW)], cidx)
        pltpu.sync_copy(p_hbm.at[pl.ds(base, BPW)], pidx)
        pltpu.sync_copy(n_hbm.at[pl.ds(base * K, NIDX)], nidx)

        @pl.loop(0, NBLK)
        def _(blk):
            off = blk * W
            cps = [
                pltpu.async_copy(in_hbm.at[cidx.at[pl.ds(off, W)]], crows,
                                 sem),
                pltpu.async_copy(out_hbm.at[pidx.at[pl.ds(off, W)]], prows,
                                 sem),
            ]
            for j in range(NEG_CH):
                cps.append(pltpu.async_copy(
                    out_hbm.at[nidx.at[pl.ds(off * K + j * GCHUNK, GCHUNK)]],
                    nrows.at[pl.ds(j * GCHUNK, GCHUNK)], sem))
            for c in cps:
                c.wait()

            iota = lax.iota(jnp.int32, L)
            for g in range(W // L):
                rows = g * L + iota
                rows_k = rows * K

                for k0 in range(0, K, KSPLIT):
                    kspan = list(range(k0, min(k0 + KSPLIT, K)))
                    nrowv = [rows_k + kk for kk in kspan]

                    def dbody(d, accs, kspan=kspan, nrowv=nrowv):
                        dv = (d + iota) & (DW - 1)
                        cbf = plsc.bitcast(
                            plsc.load_gather(crows, [rows, dv]),
                            jnp.bfloat16)
                        new = []
                        if kspan[0] == 0:
                            pbf = plsc.bitcast(
                                plsc.load_gather(prows, [rows, dv]),
                                jnp.bfloat16)
                            pa, pb = plsc.unpack(
                                cbf * pbf, format=PF,
                                preferred_element_type=jnp.float32)
                            new.append(accs[0] + (pa + pb))
                        for j, kk in enumerate(kspan):
                            nbf = plsc.bitcast(
                                plsc.load_gather(nrows, [nrowv[j], dv]),
                                jnp.bfloat16)
                            pa, pb = plsc.unpack(
                                cbf * nbf, format=PF,
                                preferred_element_type=jnp.float32)
                            new.append(accs[len(new)] + (pa + pb))
                        return tuple(new)

                    n_acc = len(kspan) + (1 if k0 == 0 else 0)
                    init = tuple(jnp.zeros((L,), jnp.float32)
                                 for _ in range(n_acc))
                    res = lax.fori_loop(0, DW, dbody, init)
                    if k0 == 0:
                        poss[pl.ds(off + g * L, L)] = res[0]
                        res = res[1:]
                    for j, kk in enumerate(kspan):
                        negs[pl.ds(kk * BPW + off + g * L, L)] = res[j]

        pltpu.sync_copy(poss, pos_out.at[pl.ds(base, BPW)])
        pltpu.sync_copy(negs, neg_out.at[pl.ds(base * K, NIDX)])

    return k(in_lin, out_lin, center, positive, negatives_flat)


def _tc_loss(pos_s, neg_s, B, K):
    pos2 = pos_s.reshape(B // 128, 128)
    neg2 = neg_s.reshape((B * K) // 128, 128)

    def body(pos_ref, neg_ref, o_ref):
        ps = pos_ref[...]
        ns = neg_ref[...]
        pos_loss = jnp.sum(jnp.maximum(-ps, 0.0)
                           + jnp.log1p(jnp.exp(-jnp.abs(ps))))
        neg_loss = jnp.sum(jnp.maximum(ns, 0.0)
                           + jnp.log1p(jnp.exp(-jnp.abs(ns))))
        o_ref[0, 0] = pos_loss / B + neg_loss / (B * K)

    return pl.pallas_call(
        body,
        out_shape=jax.ShapeDtypeStruct((1, 1), jnp.float32),
        in_specs=[
            pl.BlockSpec(memory_space=pltpu.VMEM),
            pl.BlockSpec(memory_space=pltpu.VMEM),
        ],
        out_specs=pl.BlockSpec(memory_space=pltpu.SMEM),
    )(pos2, neg2)


def kernel(in_emb, out_emb, center, positive, negatives):
    B, K = negatives.shape
    V = in_emb.shape[0]
    ntail = V % 128
    tail_in = in_emb[V - ntail:, :].reshape(ntail * D)
    tail_out = out_emb[V - ntail:, :].reshape(ntail * D)
    lin_in, lin_out = _sc_linearize(in_emb.T, out_emb.T, tail_in, tail_out,
                                    V)
    in2 = lin_in.reshape(V, DW)
    out2 = lin_out.reshape(V, DW)
    center = center.astype(jnp.int32)
    positive = positive.astype(jnp.int32)
    negatives_flat = negatives.astype(jnp.int32).reshape(B * K)
    pos_s, neg_s = _sc_scores(in2, out2, center, positive,
                              negatives_flat, B, K)
    return _tc_loss(pos_s, neg_s, B, K)[0, 0]
